# Initial kernel scaffold; baseline (speedup 1.0000x reference)
import jax
import jax.numpy as jnp
from jax.experimental import pallas as pl


def kernel(edge_index, xs_0, xs_1, xs_2):
    raise NotImplementedError("write your pallas kernel here")


# speedup vs baseline: 1.0024x; 1.0024x over previous
import jax
import jax.numpy as jnp
from jax.experimental import pallas as pl


def _mean3_body(x0_ref, x1_ref, x2_ref, o_ref):
    o_ref[...] = (x0_ref[...] + x1_ref[...] + x2_ref[...]) * (1.0 / 3.0)


def kernel(edge_index, xs_0, xs_1, xs_2):
    n, d = xs_0.shape
    block = 4000
    while n % block != 0:
        block //= 2
    grid = (n // block,)
    spec = pl.BlockSpec((block, d), lambda i: (i, 0))
    return pl.pallas_call(
        _mean3_body,
        grid=grid,
        in_specs=[spec, spec, spec],
        out_specs=spec,
        out_shape=jax.ShapeDtypeStruct((n, d), xs_0.dtype),
    )(xs_0, xs_1, xs_2)
